# Initial kernel scaffold; baseline (speedup 1.0000x reference)
import jax
import jax.numpy as jnp
from jax.experimental import pallas as pl


def kernel(features, edge_index, emb_table, W0, b0, gamma0, beta0, W1, b1, gamma1, beta1, Wp, bp):
    raise NotImplementedError("write your pallas kernel here")


# speedup vs baseline: 13.2547x; 13.2547x over previous
import functools

import jax
import jax.numpy as jnp
from jax import lax
from jax.experimental import pallas as pl
from jax.experimental.pallas import tpu as pltpu
from jax.experimental.pallas import tpu_sc as plsc

N = 99999
NP = 100096
E = 1599984
EP = 1605632
ROWS = EP // 128
RPS = ROWS // 16
WIN = 8
NWIN = RPS // WIN
STRIPE = NP // 16
NSCRATCH = NP - N

_mesh = plsc.VectorSubcoreMesh(core_axis_name="c", subcore_axis_name="s")


@functools.partial(
    pl.kernel,
    out_type=jax.ShapeDtypeStruct((2, NP), jnp.float32),
    mesh=_mesh,
    scratch_types=[
        pltpu.VMEM((WIN, 128), jnp.int32),
        pltpu.VMEM((128,), jnp.float32),
        pltpu.VMEM((STRIPE,), jnp.float32),
        pltpu.VMEM_SHARED((NP,), jnp.float32),
    ],
    compiler_params=pltpu.CompilerParams(use_tc_tiling_on_sc=False),
    name="sc_degrees",
)
def _sc_degrees(edges_hbm, out_hbm, idx_v, ones_v, zbuf_v, deg_sp):
    c = lax.axis_index("c")
    s = lax.axis_index("s")
    for j in range(128 // 16):
        ones_v[pl.ds(j * 16, 16)] = jnp.ones((16,), jnp.float32)

    def zb(i, carry):
        zbuf_v[pl.ds(i * 16, 16)] = jnp.zeros((16,), jnp.float32)
        return carry

    lax.fori_loop(0, STRIPE // 16, zb, 0)
    pltpu.sync_copy(zbuf_v, deg_sp.at[pl.ds(s * STRIPE, STRIPE)])
    plsc.subcore_barrier()

    base = s * RPS

    def win(w, carry):
        pltpu.sync_copy(edges_hbm.at[c, pl.ds(base + w * WIN, WIN)], idx_v)
        for j in range(WIN):
            pltpu.sync_copy(ones_v, deg_sp.at[idx_v.at[j]], add=True)
        return carry

    lax.fori_loop(0, NWIN, win, 0)
    plsc.subcore_barrier()
    pltpu.sync_copy(deg_sp.at[pl.ds(s * STRIPE, STRIPE)],
                    out_hbm.at[c, pl.ds(s * STRIPE, STRIPE)])


@functools.partial(
    pl.kernel,
    out_type=jax.ShapeDtypeStruct((2, NP, 16), jnp.float32),
    mesh=_mesh,
    scratch_types=[
        pltpu.VMEM((WIN, 128), jnp.int32),
        pltpu.VMEM((WIN, 128), jnp.int32),
        pltpu.VMEM((WIN * 128, 16), jnp.float32),
        pltpu.VMEM((STRIPE // 8, 16), jnp.float32),
        pltpu.SemaphoreType.DMA,
        pltpu.VMEM_SHARED((NP, 16), jnp.float32),
    ],
    compiler_params=pltpu.CompilerParams(use_tc_tiling_on_sc=False),
    name="sc_edge_layer",
)
def _sc_layer(hflat_hbm, srcx_hbm, dst_hbm, out_hbm,
              src_v, dst_v, rows_v, zbuf_v, sem, agg_sp):
    c = lax.axis_index("c")
    s = lax.axis_index("s")

    def zb(i, carry):
        zbuf_v[i, :] = jnp.zeros((16,), jnp.float32)
        return carry

    lax.fori_loop(0, STRIPE // 8, zb, 0)
    for t in range(8):
        pltpu.sync_copy(
            zbuf_v, agg_sp.at[pl.ds(s * STRIPE + t * (STRIPE // 8), STRIPE // 8)])
    plsc.subcore_barrier()

    base = s * RPS

    def win(w, carry):
        pltpu.sync_copy(srcx_hbm.at[c, pl.ds(base + w * WIN, WIN)], src_v)
        pltpu.sync_copy(dst_hbm.at[pl.ds(base + w * WIN, WIN)], dst_v)
        copies = [
            pltpu.async_copy(hflat_hbm.at[src_v.at[j]],
                             rows_v.at[pl.ds(j * 128, 128)], sem)
            for j in range(WIN)
        ]
        for cp in copies:
            cp.wait()
        for j in range(WIN):
            pltpu.sync_copy(rows_v.at[pl.ds(j * 128, 128)],
                            agg_sp.at[dst_v.at[j]], add=True)
        return carry

    lax.fori_loop(0, NWIN, win, 0)
    plsc.subcore_barrier()
    pltpu.sync_copy(agg_sp.at[pl.ds(s * STRIPE, STRIPE)],
                    out_hbm.at[c, pl.ds(s * STRIPE, STRIPE)])


_B = 1024
_GRID = pl.cdiv(NP, _B)


def _norm(deg):
    return lax.rsqrt(jnp.maximum(deg, 1.0))


def _tc_embed_body(feat_ref, dego_ref, em_ref, w0_ref, out_ref):
    feat = feat_ref[...]
    oh = (feat == lax.broadcasted_iota(jnp.int32, (_B, 8), 1)).astype(jnp.float32)
    ew = jnp.dot(em_ref[...], w0_ref[...], preferred_element_type=jnp.float32)
    h = jnp.dot(oh, ew, preferred_element_type=jnp.float32)
    h = h * _norm(dego_ref[...])
    out_ref[0, :, :] = h[:, :16]
    out_ref[1, :, :] = h[:, 16:]


def _tc_mid_body(agg_ref, degi_ref, dego_ref, b_ref, g_ref, be_ref, w1_ref,
                 out_ref):
    a = jnp.concatenate([agg_ref[0, :, :], agg_ref[1, :, :]], axis=-1)
    a = a * _norm(degi_ref[...]) + b_ref[...]
    m = jnp.mean(a, axis=-1, keepdims=True)
    d = a - m
    v = jnp.mean(d * d, axis=-1, keepdims=True)
    a = d * lax.rsqrt(v + 1e-5) * g_ref[...] + be_ref[...]
    a = jnp.maximum(a, 0.0)
    h = jnp.dot(a * _norm(dego_ref[...]), w1_ref[...],
                preferred_element_type=jnp.float32)
    out_ref[0, :, :] = h[:, :16]
    out_ref[1, :, :] = h[:, 16:]


def _tc_fin_body(agg_ref, degi_ref, b_ref, g_ref, be_ref, wp_ref, bp_ref,
                 out_ref):
    a = jnp.concatenate([agg_ref[0, :, :], agg_ref[1, :, :]], axis=-1)
    a = a * _norm(degi_ref[...]) + b_ref[...]
    m = jnp.mean(a, axis=-1, keepdims=True)
    d = a - m
    v = jnp.mean(d * d, axis=-1, keepdims=True)
    a = d * lax.rsqrt(v + 1e-5) * g_ref[...] + be_ref[...]
    a = jnp.maximum(a, 0.0)
    out_ref[...] = jnp.dot(a, wp_ref[...],
                           preferred_element_type=jnp.float32) + bp_ref[...]


def _full(shape):
    return pl.BlockSpec(shape, lambda i: tuple(0 for _ in shape))


_tc_embed = pl.pallas_call(
    _tc_embed_body,
    grid=(_GRID,),
    in_specs=[
        pl.BlockSpec((_B, 1), lambda i: (i, 0)),
        pl.BlockSpec((_B, 1), lambda i: (i, 0)),
        _full((8, 32)),
        _full((32, 32)),
    ],
    out_specs=pl.BlockSpec((2, _B, 16), lambda i: (0, i, 0)),
    out_shape=jax.ShapeDtypeStruct((2, NP, 16), jnp.float32),
)

_tc_mid = pl.pallas_call(
    _tc_mid_body,
    grid=(_GRID,),
    in_specs=[
        pl.BlockSpec((2, _B, 16), lambda i: (0, i, 0)),
        pl.BlockSpec((_B, 1), lambda i: (i, 0)),
        pl.BlockSpec((_B, 1), lambda i: (i, 0)),
        _full((1, 32)),
        _full((1, 32)),
        _full((1, 32)),
        _full((32, 32)),
    ],
    out_specs=pl.BlockSpec((2, _B, 16), lambda i: (0, i, 0)),
    out_shape=jax.ShapeDtypeStruct((2, NP, 16), jnp.float32),
)

_NF = 11136
_BF = 128

_tc_fin = pl.pallas_call(
    _tc_fin_body,
    grid=(_NF // _BF,),
    in_specs=[
        pl.BlockSpec((2, _BF, 16), lambda i: (0, i, 0)),
        pl.BlockSpec((_BF, 1), lambda i: (i, 0)),
        _full((1, 32)),
        _full((1, 32)),
        _full((1, 32)),
        _full((32, 1)),
        _full((1, 1)),
    ],
    out_specs=pl.BlockSpec((_BF, 1), lambda i: (i, 0)),
    out_shape=jax.ShapeDtypeStruct((_NF, 1), jnp.float32),
)


def kernel(features, edge_index, emb_table, W0, b0, gamma0, beta0,
           W1, b1, gamma1, beta1, Wp, bp):
    src = edge_index[0].astype(jnp.int32)
    dst = edge_index[1].astype(jnp.int32)
    pidx = jnp.arange(EP - E, dtype=jnp.int32)
    src_p = jnp.concatenate([src, N + pidx % NSCRATCH])
    dst_p = jnp.concatenate([dst, N + (pidx * 7) % NSCRATCH])

    edge2d = jnp.stack([src_p, dst_p]).reshape(2, ROWS, 128)
    srcx = jnp.stack([src_p, src_p + NP]).reshape(2, ROWS, 128)
    dst2d = dst_p.reshape(ROWS, 128)

    deg = _sc_degrees(edge2d)
    deg_out = deg[0].reshape(NP, 1)
    deg_in = deg[1].reshape(NP, 1)

    featp = jnp.concatenate(
        [features.reshape(-1).astype(jnp.int32),
         jnp.zeros((NP - N,), jnp.int32)]).reshape(NP, 1)
    empad = jnp.concatenate([emb_table, jnp.zeros((2, 32), jnp.float32)])

    h0 = _tc_embed(featp, deg_out, empad, W0)
    agg0 = _sc_layer(h0.reshape(2 * NP, 16), srcx, dst2d)

    h1 = _tc_mid(agg0, deg_in, deg_out, b0.reshape(1, 32),
                 gamma0.reshape(1, 32), beta0.reshape(1, 32), W1)
    agg1 = _sc_layer(h1.reshape(2 * NP, 16), srcx, dst2d)

    aggf = agg1[:, :N, :].reshape(2, 11111, 9, 16)[:, :, 0, :]
    aggf = jnp.concatenate(
        [aggf, jnp.zeros((2, _NF - 11111, 16), jnp.float32)], axis=1)
    degf = jnp.concatenate(
        [deg_in[:N:9], jnp.ones((_NF - 11111, 1), jnp.float32)])

    y = _tc_fin(aggf, degf, b1.reshape(1, 32), gamma1.reshape(1, 32),
                beta1.reshape(1, 32), Wp, bp.reshape(1, 1))
    return y[:11111]


# speedup vs baseline: 22.6049x; 1.7054x over previous
import functools

import jax
import jax.numpy as jnp
from jax import lax
from jax.experimental import pallas as pl
from jax.experimental.pallas import tpu as pltpu
from jax.experimental.pallas import tpu_sc as plsc

N = 99999
NP = 100096
NSCR = NP - N
NPB = 102400
E = 1599984
EP = 1605632
ROWS = EP // 128
RPS = ROWS // 16
WIN = 8
NWIN = RPS // WIN
STRIPE = NP // 16
NG = 11111
NF = 12288
FPS = NF // 16 // 128

_scp = pltpu.CompilerParams(use_tc_tiling_on_sc=False)


def _sc_degrees_body(edges_hbm, fidx_hbm, deg_hbm, degf_hbm,
                     idx_v, ones_v, zbuf_v, fidx_v, frow_v, sem, deg_sp):
    c = lax.axis_index("c")
    s = lax.axis_index("s")
    for j in range(128 // 16):
        ones_v[pl.ds(j * 16, 16)] = jnp.ones((16,), jnp.float32)

    def zb(i, carry):
        zbuf_v[pl.ds(i * 16, 16)] = jnp.zeros((16,), jnp.float32)
        return carry

    lax.fori_loop(0, STRIPE // 16, zb, 0)
    pltpu.sync_copy(zbuf_v, deg_sp.at[pl.ds(s * STRIPE, STRIPE)])
    plsc.subcore_barrier()

    base = s * RPS

    def win(w, carry):
        pltpu.sync_copy(edges_hbm.at[c, pl.ds(base + w * WIN, WIN)], idx_v)
        for j in range(WIN):
            pltpu.sync_copy(ones_v, deg_sp.at[idx_v.at[j]], add=True)
        return carry

    lax.fori_loop(0, NWIN, win, 0)
    plsc.subcore_barrier()
    pltpu.sync_copy(deg_sp.at[pl.ds(s * STRIPE, STRIPE)],
                    deg_hbm.at[c, pl.ds(s * STRIPE, STRIPE)])

    @pl.when(c == 1)
    def _():
        pltpu.sync_copy(fidx_hbm.at[pl.ds(s * FPS, FPS)], fidx_v)
        copies = [
            pltpu.async_copy(deg_sp.at[fidx_v.at[j]],
                             frow_v.at[pl.ds(j * 128, 128)], sem)
            for j in range(FPS)
        ]
        for cp in copies:
            cp.wait()
        pltpu.sync_copy(frow_v, degf_hbm.at[pl.ds(s * (FPS * 128), FPS * 128)])


def _sc_layer_common(c, s, h_hbm, edges_hbm, src_v, dst_v, rows_v,
                     sem, agg_sp):
    def zb(i, carry):
        rows_v[i, :] = jnp.zeros((16,), jnp.float32)
        return carry

    lax.fori_loop(0, STRIPE // 8, zb, 0)
    for t in range(8):
        pltpu.sync_copy(
            rows_v.at[pl.ds(0, STRIPE // 8)],
            agg_sp.at[pl.ds(s * STRIPE + t * (STRIPE // 8), STRIPE // 8)])
    plsc.subcore_barrier()

    base = s * RPS

    def win(w, carry):
        pltpu.sync_copy(edges_hbm.at[0, pl.ds(base + w * WIN, WIN)], src_v)
        pltpu.sync_copy(edges_hbm.at[1, pl.ds(base + w * WIN, WIN)], dst_v)
        copies = [
            pltpu.async_copy(h_hbm.at[c].at[src_v.at[j]],
                             rows_v.at[pl.ds(j * 128, 128)], sem)
            for j in range(WIN)
        ]
        for cp in copies:
            cp.wait()
        for j in range(WIN):
            pltpu.sync_copy(rows_v.at[pl.ds(j * 128, 128)],
                            agg_sp.at[dst_v.at[j]], add=True)
        return carry

    lax.fori_loop(0, NWIN, win, 0)
    plsc.subcore_barrier()


def _sc_layer_full_body(h_hbm, edges_hbm, out_hbm,
                        src_v, dst_v, rows_v, sem, agg_sp):
    c = lax.axis_index("c")
    s = lax.axis_index("s")
    _sc_layer_common(c, s, h_hbm, edges_hbm, src_v, dst_v, rows_v,
                     sem, agg_sp)
    pltpu.sync_copy(agg_sp.at[pl.ds(s * STRIPE, STRIPE)],
                    out_hbm.at[c, pl.ds(s * STRIPE, STRIPE)])


def _sc_layer_firsts_body(h_hbm, edges_hbm, fidx_hbm, out_hbm,
                          src_v, dst_v, rows_v, sem, agg_sp):
    c = lax.axis_index("c")
    s = lax.axis_index("s")
    _sc_layer_common(c, s, h_hbm, edges_hbm, src_v, dst_v, rows_v,
                     sem, agg_sp)
    pltpu.sync_copy(fidx_hbm.at[pl.ds(s * FPS, FPS)], src_v.at[pl.ds(0, FPS)])
    copies = [
        pltpu.async_copy(agg_sp.at[src_v.at[j]],
                         rows_v.at[pl.ds(j * 128, 128)], sem)
        for j in range(FPS)
    ]
    for cp in copies:
        cp.wait()
    pltpu.sync_copy(rows_v.at[pl.ds(0, FPS * 128)],
                    out_hbm.at[c, pl.ds(s * (FPS * 128), FPS * 128)])


@functools.cache
def _sc_kernels():
    mesh = plsc.VectorSubcoreMesh(core_axis_name="c", subcore_axis_name="s",
                                  num_cores=2, num_subcores=16)
    sc_degrees = pl.kernel(
        _sc_degrees_body,
        out_type=(jax.ShapeDtypeStruct((2, NP), jnp.float32),
                  jax.ShapeDtypeStruct((NF,), jnp.float32)),
        mesh=mesh,
        scratch_types=[
            pltpu.VMEM((WIN, 128), jnp.int32),
            pltpu.VMEM((128,), jnp.float32),
            pltpu.VMEM((STRIPE,), jnp.float32),
            pltpu.VMEM((FPS, 128), jnp.int32),
            pltpu.VMEM((FPS * 128,), jnp.float32),
            pltpu.SemaphoreType.DMA,
            pltpu.VMEM_SHARED((NP,), jnp.float32),
        ],
        compiler_params=_scp,
        name="sc_degrees",
    )
    sc_layer_full = pl.kernel(
        _sc_layer_full_body,
        out_type=jax.ShapeDtypeStruct((2, NPB, 16), jnp.float32),
        mesh=mesh,
        scratch_types=[
            pltpu.VMEM((WIN, 128), jnp.int32),
            pltpu.VMEM((WIN, 128), jnp.int32),
            pltpu.VMEM((WIN * 128, 16), jnp.float32),
            pltpu.SemaphoreType.DMA,
            pltpu.VMEM_SHARED((NP, 16), jnp.float32),
        ],
        compiler_params=_scp,
        name="sc_edge_layer",
    )
    sc_layer_firsts = pl.kernel(
        _sc_layer_firsts_body,
        out_type=jax.ShapeDtypeStruct((2, NF, 16), jnp.float32),
        mesh=mesh,
        scratch_types=[
            pltpu.VMEM((WIN, 128), jnp.int32),
            pltpu.VMEM((WIN, 128), jnp.int32),
            pltpu.VMEM((WIN * 128, 16), jnp.float32),
            pltpu.SemaphoreType.DMA,
            pltpu.VMEM_SHARED((NP, 16), jnp.float32),
        ],
        compiler_params=_scp,
        name="sc_edge_layer_firsts",
    )
    return sc_degrees, sc_layer_full, sc_layer_firsts


_PB = 1280
_GRID = (NPB * 16 // 128) // _PB
_FB = 128
_FGRID = (NF * 16 // 128) // _FB


def _expand(d):
    rr = d.shape[0]
    j = lax.broadcasted_iota(jnp.int32, (128, 128), 0)
    l16 = lax.broadcasted_iota(jnp.int32, (128, 128), 1) // 16
    parts = []
    for k in range(16):
        m = (j == 8 * k + l16).astype(jnp.float32)
        parts.append(jnp.dot(d, m, preferred_element_type=jnp.float32)[:, None, :])
    return jnp.concatenate(parts, axis=1).reshape(rr * 16, 128)


def _seg():
    r = lax.broadcasted_iota(jnp.int32, (128, 128), 0) // 16
    c = lax.broadcasted_iota(jnp.int32, (128, 128), 1) // 16
    return (r == c).astype(jnp.float32)


def _rnorm(deg_line):
    return lax.rsqrt(jnp.maximum(_expand(deg_line), 1.0))


def _tc_embed_body(feat_ref, dego_ref, em_ref, w0_ref, out_ref):
    featv = _expand(feat_ref[...])
    nv = _rnorm(dego_ref[...])
    ew = jnp.dot(em_ref[...], w0_ref[...], preferred_element_type=jnp.float32)
    hlo = jnp.zeros((_PB, 128), jnp.float32)
    hhi = jnp.zeros((_PB, 128), jnp.float32)
    for v in range(6):
        m = (featv == float(v)).astype(jnp.float32)
        lo = jnp.concatenate([ew[v:v + 1, 0:16]] * 8, axis=1)
        hi = jnp.concatenate([ew[v:v + 1, 16:32]] * 8, axis=1)
        hlo = hlo + m * lo
        hhi = hhi + m * hi
    out_ref[0] = hlo * nv
    out_ref[1] = hhi * nv


def _ln_relu(alo, ahi, nvd, b_ref, g_ref, be_ref):
    s = _seg()
    alo = alo * nvd + b_ref[0:1, :]
    ahi = ahi * nvd + b_ref[1:2, :]
    m = jnp.dot(alo + ahi, s, preferred_element_type=jnp.float32) * (1.0 / 32.0)
    dlo = alo - m
    dhi = ahi - m
    v = jnp.dot(dlo * dlo + dhi * dhi, s,
                preferred_element_type=jnp.float32) * (1.0 / 32.0)
    r = lax.rsqrt(v + 1e-5)
    ylo = jnp.maximum(dlo * r * g_ref[0:1, :] + be_ref[0:1, :], 0.0)
    yhi = jnp.maximum(dhi * r * g_ref[1:2, :] + be_ref[1:2, :], 0.0)
    return ylo, yhi


def _tc_mid_body(agg_ref, degi_ref, dego_ref, w4_ref, b_ref, g_ref, be_ref,
                 out_ref):
    nvd = _rnorm(degi_ref[...])
    ylo, yhi = _ln_relu(agg_ref[0], agg_ref[1], nvd, b_ref, g_ref, be_ref)
    nvs = _rnorm(dego_ref[...])
    zlo = ylo * nvs
    zhi = yhi * nvs
    out_ref[0] = (jnp.dot(zlo, w4_ref[0], preferred_element_type=jnp.float32)
                  + jnp.dot(zhi, w4_ref[1], preferred_element_type=jnp.float32))
    out_ref[1] = (jnp.dot(zlo, w4_ref[2], preferred_element_type=jnp.float32)
                  + jnp.dot(zhi, w4_ref[3], preferred_element_type=jnp.float32))


def _tc_fin_body(f_ref, degf_ref, b_ref, g_ref, be_ref, wp_ref, bp_ref,
                 out_ref):
    nvd = _rnorm(degf_ref[...])
    ylo, yhi = _ln_relu(f_ref[0], f_ref[1], nvd, b_ref, g_ref, be_ref)
    s = _seg()
    ssum = jnp.dot(ylo * wp_ref[0:1, :] + yhi * wp_ref[1:2, :], s,
                   preferred_element_type=jnp.float32)
    l = lax.broadcasted_iota(jnp.int32, (128, 8), 0)
    p = lax.broadcasted_iota(jnp.int32, (128, 8), 1)
    comp = (l == 16 * p).astype(jnp.float32)
    out_ref[...] = jnp.dot(ssum, comp,
                           preferred_element_type=jnp.float32) + bp_ref[...]


def _full(shape):
    return pl.BlockSpec(shape, lambda i: tuple(0 for _ in shape))


_tc_embed = pl.pallas_call(
    _tc_embed_body,
    grid=(_GRID,),
    in_specs=[
        pl.BlockSpec((_PB // 16, 128), lambda i: (i, 0)),
        pl.BlockSpec((_PB // 16, 128), lambda i: (i, 0)),
        _full((8, 32)),
        _full((32, 32)),
    ],
    out_specs=pl.BlockSpec((2, _PB, 128), lambda i: (0, i, 0)),
    out_shape=jax.ShapeDtypeStruct((2, NPB * 16 // 128, 128), jnp.float32),
)

_tc_mid = pl.pallas_call(
    _tc_mid_body,
    grid=(_GRID,),
    in_specs=[
        pl.BlockSpec((2, _PB, 128), lambda i: (0, i, 0)),
        pl.BlockSpec((_PB // 16, 128), lambda i: (i, 0)),
        pl.BlockSpec((_PB // 16, 128), lambda i: (i, 0)),
        _full((4, 128, 128)),
        _full((2, 128)),
        _full((2, 128)),
        _full((2, 128)),
    ],
    out_specs=pl.BlockSpec((2, _PB, 128), lambda i: (0, i, 0)),
    out_shape=jax.ShapeDtypeStruct((2, NPB * 16 // 128, 128), jnp.float32),
)

_tc_fin = pl.pallas_call(
    _tc_fin_body,
    grid=(_FGRID,),
    in_specs=[
        pl.BlockSpec((2, _FB, 128), lambda i: (0, i, 0)),
        pl.BlockSpec((_FB // 16, 128), lambda i: (i, 0)),
        _full((2, 128)),
        _full((2, 128)),
        _full((2, 128)),
        _full((2, 128)),
        _full((1, 1)),
    ],
    out_specs=pl.BlockSpec((_FB, 8), lambda i: (i, 0)),
    out_shape=jax.ShapeDtypeStruct((NF * 16 // 128, 8), jnp.float32),
)


def kernel(features, edge_index, emb_table, W0, b0, gamma0, beta0,
           W1, b1, gamma1, beta1, Wp, bp):
    f32 = jnp.float32
    src = edge_index[0].astype(jnp.int32)
    dst = edge_index[1].astype(jnp.int32)
    pidx = jnp.arange(EP - E, dtype=jnp.int32)
    edges = jnp.stack([
        jnp.concatenate([src, N + pidx % NSCR]),
        jnp.concatenate([dst, N + (pidx * 7) % NSCR]),
    ]).reshape(2, ROWS, 128)

    g = jnp.arange(NF, dtype=jnp.int32)
    fidx = jnp.where(g < NG, g * 9, N + g % NSCR).reshape(NF // 128, 128)

    _sc_degrees, _sc_layer_full, _sc_layer_firsts = _sc_kernels()
    deg, degf = _sc_degrees(edges, fidx)
    degp = jnp.concatenate([deg, jnp.ones((2, NPB - NP), f32)], axis=1)
    dego_line = degp[0].reshape(NPB // 128, 128)
    degi_line = degp[1].reshape(NPB // 128, 128)

    featf = jnp.concatenate([
        features.reshape(-1).astype(f32), jnp.zeros((NPB - N,), f32)
    ]).reshape(NPB // 128, 128)
    empad = jnp.concatenate([emb_table, jnp.zeros((2, 32), f32)])

    h0 = _tc_embed(featf, dego_line, empad, W0)
    agg0 = _sc_layer_full(h0.reshape(2, NPB, 16), edges)

    eye8 = jnp.eye(8, dtype=f32)
    w4 = jnp.stack([
        jnp.kron(eye8, W1[:16, :16]), jnp.kron(eye8, W1[16:, :16]),
        jnp.kron(eye8, W1[:16, 16:]), jnp.kron(eye8, W1[16:, 16:]),
    ])

    def tiled2(v):
        return jnp.stack([jnp.tile(v[:16], 8), jnp.tile(v[16:], 8)])

    h1 = _tc_mid(agg0.reshape(2, NPB * 16 // 128, 128), degi_line, dego_line,
                 w4, tiled2(b0), tiled2(gamma0), tiled2(beta0))
    agg1 = _sc_layer_firsts(h1.reshape(2, NPB, 16), edges, fidx)

    wpt = jnp.stack([jnp.tile(Wp[:16, 0], 8), jnp.tile(Wp[16:, 0], 8)])
    y8 = _tc_fin(agg1.reshape(2, NF * 16 // 128, 128),
                 degf.reshape(NF // 128, 128), tiled2(b1), tiled2(gamma1),
                 tiled2(beta1), wpt, bp.reshape(1, 1))
    return y8.reshape(NF, 1)[:NG]
